# baseline (speedup 1.0000x reference)
import functools

import jax
import jax.numpy as jnp
from jax import lax
from jax.experimental import pallas as pl
from jax.experimental.pallas import tpu as pltpu
from jax.experimental.pallas import tpu_sc as plsc

N = 100000
E = 1600000
D_IN = 128
HID = 16
OUT = 64
LSTM_H = 32

NW = 32
LANES = 128
KB = 8
RPW = 392
R_TOT = NW * RPW
E_PAD = R_TOT * LANES
NB = RPW // KB
N_PAD = 100352
STRIPE = N_PAD // 16
ZR = 392

BLK = 4096
GRID = -(-N // BLK)

def _prop_body(h_hbm, src_hbm, dst_hbm, out_hbm, src_v, dst_v, rows_v, zv, acc, sem):
    c = lax.axis_index("c")
    s = lax.axis_index("s")
    w = s * 2 + c
    base = s * STRIPE

    def zrow(i, carry):
        zv[i] = jnp.zeros((HID,), jnp.float32)
        return carry

    lax.fori_loop(0, ZR, zrow, 0)
    for r in range(STRIPE // ZR):
        pltpu.sync_copy(zv, acc.at[pl.ds(base + r * ZR, ZR)])
    plsc.subcore_barrier()

    row0 = w * RPW

    def block(bi, carry):
        rbase = row0 + bi * KB
        pltpu.sync_copy(src_hbm.at[pl.ds(rbase, KB)], src_v)
        pltpu.sync_copy(dst_hbm.at[pl.ds(rbase, KB)], dst_v)
        copies = [
            pltpu.async_copy(h_hbm.at[src_v.at[j]], rows_v.at[j], sem)
            for j in range(KB)
        ]
        for cp in copies:
            cp.wait()
        for j in range(KB):
            pltpu.sync_copy(rows_v.at[j], acc.at[dst_v.at[j]], add=True)
        return carry

    lax.fori_loop(0, NB, block, 0)
    plsc.subcore_barrier()
    pltpu.sync_copy(acc.at[pl.ds(base, STRIPE)], out_hbm.at[c, pl.ds(base, STRIPE)])


def _deg_body(dst_hbm, out_hbm, dst_v, ones_v, zv, acc):
    c = lax.axis_index("c")
    s = lax.axis_index("s")
    w = s * 2 + c
    base = s * STRIPE

    def zchunk(i, carry):
        zv[pl.ds(i * 16, 16)] = jnp.zeros((16,), jnp.float32)
        return carry

    lax.fori_loop(0, ZR // 16, zchunk, 0)
    for j in range(LANES // 16):
        ones_v[pl.ds(j * 16, 16)] = jnp.ones((16,), jnp.float32)
    for r in range(STRIPE // ZR):
        pltpu.sync_copy(zv, acc.at[pl.ds(base + r * ZR, ZR)])
    plsc.subcore_barrier()

    row0 = w * RPW

    def block(bi, carry):
        rbase = row0 + bi * KB
        pltpu.sync_copy(dst_hbm.at[pl.ds(rbase, KB)], dst_v)
        for j in range(KB):
            pltpu.sync_copy(ones_v, acc.at[dst_v.at[j]], add=True)
        return carry

    lax.fori_loop(0, NB, block, 0)
    plsc.subcore_barrier()
    pltpu.sync_copy(acc.at[pl.ds(base, STRIPE)], out_hbm.at[c, pl.ds(base, STRIPE)])


@functools.lru_cache(maxsize=None)
def _prop_kernel():
    mesh = plsc.VectorSubcoreMesh(core_axis_name="c", subcore_axis_name="s")
    return pl.kernel(
        _prop_body,
        mesh=mesh,
        compiler_params=pltpu.CompilerParams(use_tc_tiling_on_sc=False),
        out_type=jax.ShapeDtypeStruct((2, N_PAD, HID), jnp.float32),
        scratch_types=[
            pltpu.VMEM((KB, LANES), jnp.int32),
            pltpu.VMEM((KB, LANES), jnp.int32),
            pltpu.VMEM((KB, LANES, HID), jnp.float32),
            pltpu.VMEM((ZR, HID), jnp.float32),
            pltpu.VMEM_SHARED((N_PAD, HID), jnp.float32),
            pltpu.SemaphoreType.DMA,
        ],
    )


@functools.lru_cache(maxsize=None)
def _deg_kernel():
    mesh = plsc.VectorSubcoreMesh(core_axis_name="c", subcore_axis_name="s")
    return pl.kernel(
        _deg_body,
        mesh=mesh,
        compiler_params=pltpu.CompilerParams(use_tc_tiling_on_sc=False),
        out_type=jax.ShapeDtypeStruct((2, N_PAD), jnp.float32),
        scratch_types=[
            pltpu.VMEM((KB, LANES), jnp.int32),
            pltpu.VMEM((LANES,), jnp.float32),
            pltpu.VMEM((ZR,), jnp.float32),
            pltpu.VMEM_SHARED((N_PAD,), jnp.float32),
        ],
    )


def _prop_call(h, srcr, dstr):
    return _prop_kernel()(h, srcr, dstr)


def _deg_call(dstr):
    return _deg_kernel()(dstr)


def _rowmm(a, w):
    return lax.dot_general(a, w, (((1,), (1,)), ((), ())),
                           preferred_element_type=jnp.float32)


def _k2_body(x_ref, w1_ref, degp_ref, dinv_ref, a0_ref):
    h0 = _rowmm(x_ref[...], w1_ref[...])
    deg = degp_ref[0] + degp_ref[1] + 1.0
    dinv = lax.rsqrt(deg)[:, None]
    dinv_ref[...] = dinv
    a0_ref[...] = h0 * dinv


def _k2(x, W1, degp):
    return pl.pallas_call(
        _k2_body,
        grid=(GRID,),
        in_specs=[
            pl.BlockSpec((BLK, D_IN), lambda i: (i, 0)),
            pl.BlockSpec((HID, D_IN), lambda i: (0, 0)),
            pl.BlockSpec((2, BLK), lambda i: (0, i)),
        ],
        out_specs=[
            pl.BlockSpec((BLK, 1), lambda i: (i, 0)),
            pl.BlockSpec((BLK, HID), lambda i: (i, 0)),
        ],
        out_shape=[
            jax.ShapeDtypeStruct((N, 1), jnp.float32),
            jax.ShapeDtypeStruct((N, HID), jnp.float32),
        ],
    )(x, W1, degp)


def _k3_body(p_ref, a0_ref, dinv_ref, b1_ref, w2_ref, x1_ref, a1_ref):
    dinv = dinv_ref[...]
    ssum = p_ref[0] + p_ref[1] + a0_ref[...]
    x1 = jnp.maximum(ssum * dinv + b1_ref[...], 0.0)
    x1_ref[...] = x1
    a1_ref[...] = _rowmm(x1, w2_ref[...]) * dinv


def _k3(P0, a0, dinv, b1, W2):
    return pl.pallas_call(
        _k3_body,
        grid=(GRID,),
        in_specs=[
            pl.BlockSpec((2, BLK, HID), lambda i: (0, i, 0)),
            pl.BlockSpec((BLK, HID), lambda i: (i, 0)),
            pl.BlockSpec((BLK, 1), lambda i: (i, 0)),
            pl.BlockSpec((HID,), lambda i: (0,)),
            pl.BlockSpec((HID, HID), lambda i: (0, 0)),
        ],
        out_specs=[
            pl.BlockSpec((BLK, HID), lambda i: (i, 0)),
            pl.BlockSpec((BLK, HID), lambda i: (i, 0)),
        ],
        out_shape=[
            jax.ShapeDtypeStruct((N, HID), jnp.float32),
            jax.ShapeDtypeStruct((N, HID), jnp.float32),
        ],
    )(P0, a0, dinv, b1, W2)


def _lstm_cell(xt, h, c, wih, whh, bsum):
    g = _rowmm(xt, wih) + _rowmm(h, whh) + bsum
    sg = jax.nn.sigmoid(g)
    i = sg[:, 0 * LSTM_H:1 * LSTM_H]
    f = sg[:, 1 * LSTM_H:2 * LSTM_H]
    gg = jnp.tanh(g[:, 2 * LSTM_H:3 * LSTM_H])
    o = sg[:, 3 * LSTM_H:4 * LSTM_H]
    c = f * c + i * gg
    h = o * jnp.tanh(c)
    return h, c


def _k4_body(p_ref, a1_ref, dinv_ref, b2_ref, x1_ref,
             wihf_ref, whhf_ref, bf_ref, wihr_ref, whhr_ref, br_ref,
             watt_ref, batt_ref, aj_ref):
    dinv = dinv_ref[...]
    ssum = p_ref[0] + p_ref[1] + a1_ref[...]
    x2 = jnp.maximum(ssum * dinv + b2_ref[...], 0.0)
    x1 = x1_ref[...]
    xs = (x1, x2)

    bf = bf_ref[...]
    br = br_ref[...]
    z = jnp.zeros((x1.shape[0], LSTM_H), jnp.float32)
    h, c = _lstm_cell(xs[0], z, z, wihf_ref[...], whhf_ref[...], bf)
    f0 = h
    h, c = _lstm_cell(xs[1], h, c, wihf_ref[...], whhf_ref[...], bf)
    f1 = h
    h, c = _lstm_cell(xs[1], z, z, wihr_ref[...], whhr_ref[...], br)
    r1 = h
    h, c = _lstm_cell(xs[0], h, c, wihr_ref[...], whhr_ref[...], br)
    r0 = h

    watt = watt_ref[...]
    batt = batt_ref[...]
    al0 = _rowmm(jnp.concatenate([f0, r0], axis=1), watt)[:, 0] + batt[0]
    al1 = _rowmm(jnp.concatenate([f1, r1], axis=1), watt)[:, 0] + batt[0]
    m = jnp.maximum(al0, al1)
    e0 = jnp.exp(al0 - m)
    e1 = jnp.exp(al1 - m)
    inv = 1.0 / (e0 + e1)
    xj = (x1 * (e0 * inv)[:, None] + x2 * (e1 * inv)[:, None])
    aj_ref[...] = xj * dinv


def _k4(P1, a1, dinv, b2, x1, W_ih_f, W_hh_f, bsum_f, W_ih_r, W_hh_r, bsum_r,
        W_att, b_att):
    return pl.pallas_call(
        _k4_body,
        grid=(GRID,),
        in_specs=[
            pl.BlockSpec((2, BLK, HID), lambda i: (0, i, 0)),
            pl.BlockSpec((BLK, HID), lambda i: (i, 0)),
            pl.BlockSpec((BLK, 1), lambda i: (i, 0)),
            pl.BlockSpec((HID,), lambda i: (0,)),
            pl.BlockSpec((BLK, HID), lambda i: (i, 0)),
            pl.BlockSpec((4 * LSTM_H, HID), lambda i: (0, 0)),
            pl.BlockSpec((4 * LSTM_H, LSTM_H), lambda i: (0, 0)),
            pl.BlockSpec((4 * LSTM_H,), lambda i: (0,)),
            pl.BlockSpec((4 * LSTM_H, HID), lambda i: (0, 0)),
            pl.BlockSpec((4 * LSTM_H, LSTM_H), lambda i: (0, 0)),
            pl.BlockSpec((4 * LSTM_H,), lambda i: (0,)),
            pl.BlockSpec((1, 2 * LSTM_H), lambda i: (0, 0)),
            pl.BlockSpec((1,), lambda i: (0,)),
        ],
        out_specs=pl.BlockSpec((BLK, HID), lambda i: (i, 0)),
        out_shape=jax.ShapeDtypeStruct((N, HID), jnp.float32),
    )(P1, a1, dinv, b2, x1, W_ih_f, W_hh_f, bsum_f, W_ih_r, W_hh_r, bsum_r,
      W_att, b_att)


def _k5_body(p_ref, aj_ref, dinv_ref, wlin_ref, blin_ref, out_ref):
    dinv = dinv_ref[...]
    xp = (p_ref[0] + p_ref[1] + aj_ref[...]) * dinv
    o = _rowmm(xp, wlin_ref[...]) + blin_ref[...]
    m = jnp.max(o, axis=1, keepdims=True)
    zz = o - m
    lse = jnp.log(jnp.sum(jnp.exp(zz), axis=1, keepdims=True))
    out_ref[...] = zz - lse


def _k5(Pj, aj, dinv, W_lin, b_lin):
    return pl.pallas_call(
        _k5_body,
        grid=(GRID,),
        in_specs=[
            pl.BlockSpec((2, BLK, HID), lambda i: (0, i, 0)),
            pl.BlockSpec((BLK, HID), lambda i: (i, 0)),
            pl.BlockSpec((BLK, 1), lambda i: (i, 0)),
            pl.BlockSpec((OUT, HID), lambda i: (0, 0)),
            pl.BlockSpec((OUT,), lambda i: (0,)),
        ],
        out_specs=pl.BlockSpec((BLK, OUT), lambda i: (i, 0)),
        out_shape=jax.ShapeDtypeStruct((N, OUT), jnp.float32),
    )(Pj, aj, dinv, W_lin, b_lin)


def kernel(x, edge_index, W1, b1, W2, b2, W_ih_f, W_hh_f, b_ih_f, b_hh_f,
           W_ih_r, W_hh_r, b_ih_r, b_hh_r, W_att, b_att, W_lin, b_lin):
    src = edge_index[0]
    dst = edge_index[1]
    pad_src = jnp.zeros((E_PAD - E,), jnp.int32)
    pad_dst = jnp.full((E_PAD - E,), N_PAD - 1, jnp.int32)
    srcr = jnp.concatenate([src, pad_src]).reshape(R_TOT, LANES)
    dstr = jnp.concatenate([dst, pad_dst]).reshape(R_TOT, LANES)

    degp = _deg_call(dstr)
    dinv, a0 = _k2(x, W1, degp)
    P0 = _prop_call(a0, srcr, dstr)
    x1, a1 = _k3(P0, a0, dinv, b1, W2)
    P1 = _prop_call(a1, srcr, dstr)
    aj = _k4(P1, a1, dinv, b2, x1, W_ih_f, W_hh_f, b_ih_f + b_hh_f,
             W_ih_r, W_hh_r, b_ih_r + b_hh_r, W_att, b_att)
    Pj = _prop_call(aj, srcr, dstr)
    return _k5(Pj, aj, dinv, W_lin, b_lin)


# speedup vs baseline: 1.2570x; 1.2570x over previous
import functools

import jax
import jax.numpy as jnp
from jax import lax
from jax.experimental import pallas as pl
from jax.experimental.pallas import tpu as pltpu
from jax.experimental.pallas import tpu_sc as plsc

N = 100000
E = 1600000
D_IN = 128
HID = 16
OUT = 64
LSTM_H = 32

NW = 32
LANES = 128
KB = 4
DKB = 8
RPW = 392
R_TOT = NW * RPW
E_PAD = R_TOT * LANES
NB = RPW // KB
DNB = RPW // DKB
N_PAD = 100352
STRIPE = N_PAD // 16
ZR = 112

BLK = 4096
GRID = -(-N // BLK)

def _prop_body(h_hbm, src_hbm, dst_hbm, out_hbm,
               sv0, dv0, rv0, sv1, dv1, rv1, zv, acc,
               isem0, isem1, gsem0, gsem1, ssem0, ssem1):
    c = lax.axis_index("c")
    s = lax.axis_index("s")
    w = s * 2 + c
    base = s * STRIPE

    sv = (sv0, sv1)
    dv = (dv0, dv1)
    rv = (rv0, rv1)
    isem = (isem0, isem1)
    gsem = (gsem0, gsem1)
    ssem = (ssem0, ssem1)

    def zrow(i, carry):
        zv[i] = jnp.zeros((HID,), jnp.float32)
        return carry

    lax.fori_loop(0, ZR, zrow, 0)
    for r in range(STRIPE // ZR):
        pltpu.sync_copy(zv, acc.at[pl.ds(base + r * ZR, ZR)])
    plsc.subcore_barrier()

    row0 = w * RPW

    def idx_start(i, b):
        rbase = row0 + i * KB
        pltpu.async_copy(src_hbm.at[pl.ds(rbase, KB)], sv[b], isem[b])
        pltpu.async_copy(dst_hbm.at[pl.ds(rbase, KB)], dv[b], isem[b])

    def idx_wait(b):
        pltpu.make_async_copy(src_hbm.at[pl.ds(0, KB)], sv[b], isem[b]).wait()
        pltpu.make_async_copy(dst_hbm.at[pl.ds(0, KB)], dv[b], isem[b]).wait()

    def gat_start(b):
        for j in range(KB):
            pltpu.async_copy(h_hbm.at[sv[b].at[j]], rv[b].at[j], gsem[b])

    def gat_wait(b):
        for j in range(KB):
            pltpu.make_async_copy(h_hbm.at[sv[b].at[j]], rv[b].at[j],
                                  gsem[b]).wait()

    def sca_start(b):
        for j in range(KB):
            pltpu.async_copy(rv[b].at[j], acc.at[dv[b].at[j]], ssem[b],
                             add=True)

    def sca_wait(b):
        for j in range(KB):
            pltpu.make_async_copy(rv[b].at[j], acc.at[dv[b].at[j]],
                                  ssem[b]).wait()

    def step(i, b, first):
        if not first:
            sca_wait(b)
        idx_start(i, b)
        gat_wait(b ^ 1)
        sca_start(b ^ 1)
        idx_wait(b)
        gat_start(b)

    idx_start(0, 0)
    idx_wait(0)
    gat_start(0)
    step(1, 1, True)

    def pair(j, carry):
        step(2 + 2 * j, 0, False)
        step(3 + 2 * j, 1, False)
        return carry

    lax.fori_loop(0, (NB - 2) // 2, pair, 0)

    gat_wait(1)
    sca_start(1)
    sca_wait(0)
    sca_wait(1)
    plsc.subcore_barrier()
    pltpu.sync_copy(acc.at[pl.ds(base, STRIPE)], out_hbm.at[c, pl.ds(base, STRIPE)])


def _deg_body(dst_hbm, out_hbm, dst_v, ones_v, zv, acc):
    c = lax.axis_index("c")
    s = lax.axis_index("s")
    w = s * 2 + c
    base = s * STRIPE

    def zchunk(i, carry):
        zv[pl.ds(i * 16, 16)] = jnp.zeros((16,), jnp.float32)
        return carry

    lax.fori_loop(0, ZR // 16, zchunk, 0)
    for j in range(LANES // 16):
        ones_v[pl.ds(j * 16, 16)] = jnp.ones((16,), jnp.float32)
    for r in range(STRIPE // ZR):
        pltpu.sync_copy(zv, acc.at[pl.ds(base + r * ZR, ZR)])
    plsc.subcore_barrier()

    row0 = w * RPW

    def block(bi, carry):
        rbase = row0 + bi * DKB
        pltpu.sync_copy(dst_hbm.at[pl.ds(rbase, DKB)], dst_v)
        for j in range(DKB):
            pltpu.sync_copy(ones_v, acc.at[dst_v.at[j]], add=True)
        return carry

    lax.fori_loop(0, DNB, block, 0)
    plsc.subcore_barrier()
    pltpu.sync_copy(acc.at[pl.ds(base, STRIPE)], out_hbm.at[c, pl.ds(base, STRIPE)])


@functools.lru_cache(maxsize=None)
def _prop_kernel():
    mesh = plsc.VectorSubcoreMesh(core_axis_name="c", subcore_axis_name="s")
    return pl.kernel(
        _prop_body,
        mesh=mesh,
        compiler_params=pltpu.CompilerParams(use_tc_tiling_on_sc=False),
        out_type=jax.ShapeDtypeStruct((2, N_PAD, HID), jnp.float32),
        scratch_types=[
            pltpu.VMEM((KB, LANES), jnp.int32),
            pltpu.VMEM((KB, LANES), jnp.int32),
            pltpu.VMEM((KB, LANES, HID), jnp.float32),
            pltpu.VMEM((KB, LANES), jnp.int32),
            pltpu.VMEM((KB, LANES), jnp.int32),
            pltpu.VMEM((KB, LANES, HID), jnp.float32),
            pltpu.VMEM((ZR, HID), jnp.float32),
            pltpu.VMEM_SHARED((N_PAD, HID), jnp.float32),
            pltpu.SemaphoreType.DMA,
            pltpu.SemaphoreType.DMA,
            pltpu.SemaphoreType.DMA,
            pltpu.SemaphoreType.DMA,
            pltpu.SemaphoreType.DMA,
            pltpu.SemaphoreType.DMA,
        ],
    )


@functools.lru_cache(maxsize=None)
def _deg_kernel():
    mesh = plsc.VectorSubcoreMesh(core_axis_name="c", subcore_axis_name="s")
    return pl.kernel(
        _deg_body,
        mesh=mesh,
        compiler_params=pltpu.CompilerParams(use_tc_tiling_on_sc=False),
        out_type=jax.ShapeDtypeStruct((2, N_PAD), jnp.float32),
        scratch_types=[
            pltpu.VMEM((DKB, LANES), jnp.int32),
            pltpu.VMEM((LANES,), jnp.float32),
            pltpu.VMEM((ZR,), jnp.float32),
            pltpu.VMEM_SHARED((N_PAD,), jnp.float32),
        ],
    )


def _prop_call(h, srcr, dstr):
    return _prop_kernel()(h, srcr, dstr)


def _deg_call(dstr):
    return _deg_kernel()(dstr)


def _mm(a, w):
    return lax.dot_general(a, w, (((1,), (0,)), ((), ())),
                           preferred_element_type=jnp.float32)


def _rowmm(a, w):
    return lax.dot_general(a, w, (((1,), (1,)), ((), ())),
                           preferred_element_type=jnp.float32)


def _k2_body(x_ref, w1_ref, degp_ref, dinv_ref, a0_ref):
    h0 = _rowmm(x_ref[...], w1_ref[...])
    deg = degp_ref[0] + degp_ref[1] + 1.0
    dinv = lax.rsqrt(deg)[:, None]
    dinv_ref[...] = dinv
    a0_ref[...] = h0 * dinv


def _k2(x, W1, degp):
    return pl.pallas_call(
        _k2_body,
        grid=(GRID,),
        in_specs=[
            pl.BlockSpec((BLK, D_IN), lambda i: (i, 0)),
            pl.BlockSpec((HID, D_IN), lambda i: (0, 0)),
            pl.BlockSpec((2, BLK), lambda i: (0, i)),
        ],
        out_specs=[
            pl.BlockSpec((BLK, 1), lambda i: (i, 0)),
            pl.BlockSpec((BLK, HID), lambda i: (i, 0)),
        ],
        out_shape=[
            jax.ShapeDtypeStruct((N, 1), jnp.float32),
            jax.ShapeDtypeStruct((N, HID), jnp.float32),
        ],
    )(x, W1, degp)


def _k3_body(p_ref, a0_ref, dinv_ref, b1_ref, w2_ref, x1_ref, a1_ref):
    dinv = dinv_ref[...]
    ssum = p_ref[0] + p_ref[1] + a0_ref[...]
    x1 = jnp.maximum(ssum * dinv + b1_ref[...], 0.0)
    x1_ref[...] = x1
    a1_ref[...] = _rowmm(x1, w2_ref[...]) * dinv


def _k3(P0, a0, dinv, b1, W2):
    return pl.pallas_call(
        _k3_body,
        grid=(GRID,),
        in_specs=[
            pl.BlockSpec((2, BLK, HID), lambda i: (0, i, 0)),
            pl.BlockSpec((BLK, HID), lambda i: (i, 0)),
            pl.BlockSpec((BLK, 1), lambda i: (i, 0)),
            pl.BlockSpec((HID,), lambda i: (0,)),
            pl.BlockSpec((HID, HID), lambda i: (0, 0)),
        ],
        out_specs=[
            pl.BlockSpec((BLK, HID), lambda i: (i, 0)),
            pl.BlockSpec((BLK, HID), lambda i: (i, 0)),
        ],
        out_shape=[
            jax.ShapeDtypeStruct((N, HID), jnp.float32),
            jax.ShapeDtypeStruct((N, HID), jnp.float32),
        ],
    )(P0, a0, dinv, b1, W2)


def _k4_body(p_ref, a1_ref, dinv_ref, b2_ref, x1_ref,
             wx1_ref, wx2_ref, bx_ref, wh_ref, wda_ref, wdb_ref, aj_ref):
    dinv = dinv_ref[...]
    ssum = p_ref[0] + p_ref[1] + a1_ref[...]
    x2 = jnp.maximum(ssum * dinv + b2_ref[...], 0.0)
    x1 = x1_ref[...]

    G = _mm(x1, wx1_ref[...]) + _mm(x2, wx2_ref[...]) + bx_ref[...]
    cA = jax.nn.sigmoid(G[:, 0:128]) * jnp.tanh(G[:, 128:256])
    hA = jax.nn.sigmoid(G[:, 256:384]) * jnp.tanh(cA)
    GB = G[:, 384:896] + _mm(hA, wh_ref[...])
    cB = (jax.nn.sigmoid(GB[:, 128:256]) * cA
          + jax.nn.sigmoid(GB[:, 0:128]) * jnp.tanh(GB[:, 256:384]))
    hB = jax.nn.sigmoid(GB[:, 384:512]) * jnp.tanh(cB)

    ad = _mm(hA, wda_ref[...]) + _mm(hB, wdb_ref[...])
    w1 = jax.nn.sigmoid(ad)
    xj = x1 + (x2 - x1) * w1
    aj_ref[...] = xj * dinv


def _k4(P1, a1, dinv, b2, x1, WX1, WX2, bX, WH, wdA, wdB):
    return pl.pallas_call(
        _k4_body,
        grid=(GRID,),
        in_specs=[
            pl.BlockSpec((2, BLK, HID), lambda i: (0, i, 0)),
            pl.BlockSpec((BLK, HID), lambda i: (i, 0)),
            pl.BlockSpec((BLK, 1), lambda i: (i, 0)),
            pl.BlockSpec((HID,), lambda i: (0,)),
            pl.BlockSpec((BLK, HID), lambda i: (i, 0)),
            pl.BlockSpec((HID, 896), lambda i: (0, 0)),
            pl.BlockSpec((HID, 896), lambda i: (0, 0)),
            pl.BlockSpec((896,), lambda i: (0,)),
            pl.BlockSpec((128, 512), lambda i: (0, 0)),
            pl.BlockSpec((128, 1), lambda i: (0, 0)),
            pl.BlockSpec((128, 1), lambda i: (0, 0)),
        ],
        out_specs=pl.BlockSpec((BLK, HID), lambda i: (i, 0)),
        out_shape=jax.ShapeDtypeStruct((N, HID), jnp.float32),
    )(P1, a1, dinv, b2, x1, WX1, WX2, bX, WH, wdA, wdB)


def _k4_weights(W_ih_f, W_hh_f, b_ih_f, b_hh_f, W_ih_r, W_hh_r, b_ih_r, b_hh_r,
                W_att):
    Tf, Tr = W_ih_f.T, W_ih_r.T
    Uf, Ur = W_hh_f.T, W_hh_r.T
    bsf = b_ih_f + b_hh_f
    bsr = b_ih_r + b_hh_r
    gi, gf, gg, go = (slice(32 * k, 32 * (k + 1)) for k in range(4))
    Z16 = jnp.zeros((HID, 32), jnp.float32)
    Z16w = jnp.zeros((HID, 64), jnp.float32)

    def xg(fwd, rev):
        return jnp.concatenate([fwd if fwd is not None else Z16,
                                rev if rev is not None else Z16, Z16w], axis=1)

    WX1 = jnp.concatenate(
        [xg(Tf[:, gi], None), xg(Tf[:, gg], None), xg(Tf[:, go], None),
         xg(None, Tr[:, gi]), xg(None, Tr[:, gf]), xg(None, Tr[:, gg]),
         xg(None, Tr[:, go])], axis=1)
    WX2 = jnp.concatenate(
        [xg(None, Tr[:, gi]), xg(None, Tr[:, gg]), xg(None, Tr[:, go]),
         xg(Tf[:, gi], None), xg(Tf[:, gf], None), xg(Tf[:, gg], None),
         xg(Tf[:, go], None)], axis=1)
    z64 = jnp.zeros((64,), jnp.float32)

    def bg(gsl):
        return jnp.concatenate([bsf[gsl], bsr[gsl], z64])

    bX = jnp.concatenate([bg(gi), bg(gg), bg(go), bg(gi), bg(gf), bg(gg),
                          bg(go)])
    Z3296 = jnp.zeros((LSTM_H, 96), jnp.float32)
    Z3232 = jnp.zeros((LSTM_H, 32), jnp.float32)
    band_f = jnp.concatenate(
        [jnp.concatenate([Uf[:, g], Z3296], axis=1) for g in (gi, gf, gg, go)],
        axis=1)
    band_r = jnp.concatenate(
        [jnp.concatenate([Z3232, Ur[:, g], Z3296[:, :64]], axis=1)
         for g in (gi, gf, gg, go)], axis=1)
    WH = jnp.concatenate([band_f, band_r,
                          jnp.zeros((64, 512), jnp.float32)], axis=0)
    w = W_att[0]
    wdA = jnp.concatenate([-w[0:32], w[32:64], z64])[:, None]
    wdB = jnp.concatenate([w[0:32], -w[32:64], z64])[:, None]
    return WX1, WX2, bX, WH, wdA, wdB


def _k5_body(p_ref, aj_ref, dinv_ref, wlin_ref, blin_ref, out_ref):
    dinv = dinv_ref[...]
    xp = (p_ref[0] + p_ref[1] + aj_ref[...]) * dinv
    o = _rowmm(xp, wlin_ref[...]) + blin_ref[...]
    m = jnp.max(o, axis=1, keepdims=True)
    zz = o - m
    lse = jnp.log(jnp.sum(jnp.exp(zz), axis=1, keepdims=True))
    out_ref[...] = zz - lse


def _k5(Pj, aj, dinv, W_lin, b_lin):
    return pl.pallas_call(
        _k5_body,
        grid=(GRID,),
        in_specs=[
            pl.BlockSpec((2, BLK, HID), lambda i: (0, i, 0)),
            pl.BlockSpec((BLK, HID), lambda i: (i, 0)),
            pl.BlockSpec((BLK, 1), lambda i: (i, 0)),
            pl.BlockSpec((OUT, HID), lambda i: (0, 0)),
            pl.BlockSpec((OUT,), lambda i: (0,)),
        ],
        out_specs=pl.BlockSpec((BLK, OUT), lambda i: (i, 0)),
        out_shape=jax.ShapeDtypeStruct((N, OUT), jnp.float32),
    )(Pj, aj, dinv, W_lin, b_lin)


def kernel(x, edge_index, W1, b1, W2, b2, W_ih_f, W_hh_f, b_ih_f, b_hh_f,
           W_ih_r, W_hh_r, b_ih_r, b_hh_r, W_att, b_att, W_lin, b_lin):
    src = edge_index[0]
    dst = edge_index[1]
    pad_src = jnp.zeros((E_PAD - E,), jnp.int32)
    pad_dst = jnp.full((E_PAD - E,), N_PAD - 1, jnp.int32)
    srcr = jnp.concatenate([src, pad_src]).reshape(R_TOT, LANES)
    dstr = jnp.concatenate([dst, pad_dst]).reshape(R_TOT, LANES)

    degp = _deg_call(dstr)
    dinv, a0 = _k2(x, W1, degp)
    P0 = _prop_call(a0, srcr, dstr)
    x1, a1 = _k3(P0, a0, dinv, b1, W2)
    P1 = _prop_call(a1, srcr, dstr)
    WX1, WX2, bX, WH, wdA, wdB = _k4_weights(
        W_ih_f, W_hh_f, b_ih_f, b_hh_f, W_ih_r, W_hh_r, b_ih_r, b_hh_r, W_att)
    aj = _k4(P1, a1, dinv, b2, x1, WX1, WX2, bX, WH, wdA, wdB)
    Pj = _prop_call(aj, srcr, dstr)
    return _k5(Pj, aj, dinv, W_lin, b_lin)


# speedup vs baseline: 1.7239x; 1.3715x over previous
import functools

import jax
import jax.numpy as jnp
from jax import lax
from jax.experimental import pallas as pl
from jax.experimental.pallas import tpu as pltpu
from jax.experimental.pallas import tpu_sc as plsc

N = 100000
E = 1600000
D_IN = 128
HID = 16
OUT = 64
LSTM_H = 32

NW = 32
LANES = 128
KB = 4
DKB = 8
RPW = 392
R_TOT = NW * RPW
E_PAD = R_TOT * LANES
NB = RPW // KB
DNB = RPW // DKB
N_PAD = 100352
STRIPE = N_PAD // 16
ZR = 112

BLK = 4096
GRID = -(-N // BLK)
PR = N_PAD // 8
PBLK = BLK // 8

def _prop_body(h_hbm, src_hbm, dst_hbm, out_hbm,
               sv0, dv0, rv0, sv1, dv1, rv1, zv, acc,
               isem0, isem1, gsem0, gsem1, ssem0, ssem1):
    c = lax.axis_index("c")
    s = lax.axis_index("s")
    w = s * 2 + c
    base = s * STRIPE

    sv = (sv0, sv1)
    dv = (dv0, dv1)
    rv = (rv0, rv1)
    isem = (isem0, isem1)
    gsem = (gsem0, gsem1)
    ssem = (ssem0, ssem1)

    def zrow(i, carry):
        zv[i] = jnp.zeros((HID,), jnp.float32)
        return carry

    lax.fori_loop(0, ZR, zrow, 0)
    for r in range(STRIPE // ZR):
        pltpu.sync_copy(zv, acc.at[pl.ds(base + r * ZR, ZR)])
    plsc.subcore_barrier()

    row0 = w * RPW

    def idx_start(i, b):
        rbase = row0 + i * KB
        pltpu.async_copy(src_hbm.at[pl.ds(rbase, KB)], sv[b], isem[b])
        pltpu.async_copy(dst_hbm.at[pl.ds(rbase, KB)], dv[b], isem[b])

    def idx_wait(b):
        pltpu.make_async_copy(src_hbm.at[pl.ds(0, KB)], sv[b], isem[b]).wait()
        pltpu.make_async_copy(dst_hbm.at[pl.ds(0, KB)], dv[b], isem[b]).wait()

    def gat_start(b):
        for j in range(KB):
            pltpu.async_copy(h_hbm.at[sv[b].at[j]], rv[b].at[j], gsem[b])

    def gat_wait(b):
        for j in range(KB):
            pltpu.make_async_copy(h_hbm.at[sv[b].at[j]], rv[b].at[j],
                                  gsem[b]).wait()

    def sca_start(b):
        for j in range(KB):
            pltpu.async_copy(rv[b].at[j], acc.at[dv[b].at[j]], ssem[b],
                             add=True)

    def sca_wait(b):
        for j in range(KB):
            pltpu.make_async_copy(rv[b].at[j], acc.at[dv[b].at[j]],
                                  ssem[b]).wait()

    def step(i, b, first):
        if not first:
            sca_wait(b)
        idx_start(i, b)
        gat_wait(b ^ 1)
        sca_start(b ^ 1)
        idx_wait(b)
        gat_start(b)

    idx_start(0, 0)
    idx_wait(0)
    gat_start(0)
    step(1, 1, True)

    def pair(j, carry):
        step(2 + 2 * j, 0, False)
        step(3 + 2 * j, 1, False)
        return carry

    lax.fori_loop(0, (NB - 2) // 2, pair, 0)

    gat_wait(1)
    sca_start(1)
    sca_wait(0)
    sca_wait(1)
    plsc.subcore_barrier()
    pltpu.sync_copy(acc.at[pl.ds(base, STRIPE)], out_hbm.at[c, pl.ds(base, STRIPE)])


def _deg_body(dst_hbm, out_hbm, dst_v, ones_v, zv, acc):
    c = lax.axis_index("c")
    s = lax.axis_index("s")
    w = s * 2 + c
    base = s * STRIPE

    def zchunk(i, carry):
        zv[pl.ds(i * 16, 16)] = jnp.zeros((16,), jnp.float32)
        return carry

    lax.fori_loop(0, ZR // 16, zchunk, 0)
    for j in range(LANES // 16):
        ones_v[pl.ds(j * 16, 16)] = jnp.ones((16,), jnp.float32)
    for r in range(STRIPE // ZR):
        pltpu.sync_copy(zv, acc.at[pl.ds(base + r * ZR, ZR)])
    plsc.subcore_barrier()

    row0 = w * RPW

    def block(bi, carry):
        rbase = row0 + bi * DKB
        pltpu.sync_copy(dst_hbm.at[pl.ds(rbase, DKB)], dst_v)
        for j in range(DKB):
            pltpu.sync_copy(ones_v, acc.at[dst_v.at[j]], add=True)
        return carry

    lax.fori_loop(0, DNB, block, 0)
    plsc.subcore_barrier()
    pltpu.sync_copy(acc.at[pl.ds(base, STRIPE)], out_hbm.at[c, pl.ds(base, STRIPE)])


@functools.lru_cache(maxsize=None)
def _prop_kernel():
    mesh = plsc.VectorSubcoreMesh(core_axis_name="c", subcore_axis_name="s")
    return pl.kernel(
        _prop_body,
        mesh=mesh,
        compiler_params=pltpu.CompilerParams(use_tc_tiling_on_sc=False),
        out_type=jax.ShapeDtypeStruct((2, N_PAD, HID), jnp.float32),
        scratch_types=[
            pltpu.VMEM((KB, LANES), jnp.int32),
            pltpu.VMEM((KB, LANES), jnp.int32),
            pltpu.VMEM((KB, LANES, HID), jnp.float32),
            pltpu.VMEM((KB, LANES), jnp.int32),
            pltpu.VMEM((KB, LANES), jnp.int32),
            pltpu.VMEM((KB, LANES, HID), jnp.float32),
            pltpu.VMEM((ZR, HID), jnp.float32),
            pltpu.VMEM_SHARED((N_PAD, HID), jnp.float32),
            pltpu.SemaphoreType.DMA,
            pltpu.SemaphoreType.DMA,
            pltpu.SemaphoreType.DMA,
            pltpu.SemaphoreType.DMA,
            pltpu.SemaphoreType.DMA,
            pltpu.SemaphoreType.DMA,
        ],
    )


@functools.lru_cache(maxsize=None)
def _deg_kernel():
    mesh = plsc.VectorSubcoreMesh(core_axis_name="c", subcore_axis_name="s")
    return pl.kernel(
        _deg_body,
        mesh=mesh,
        compiler_params=pltpu.CompilerParams(use_tc_tiling_on_sc=False),
        out_type=jax.ShapeDtypeStruct((2, N_PAD), jnp.float32),
        scratch_types=[
            pltpu.VMEM((DKB, LANES), jnp.int32),
            pltpu.VMEM((LANES,), jnp.float32),
            pltpu.VMEM((ZR,), jnp.float32),
            pltpu.VMEM_SHARED((N_PAD,), jnp.float32),
        ],
    )


def _prop_call(h, srcr, dstr):
    return _prop_kernel()(h, srcr, dstr)


def _deg_call(dstr):
    return _deg_kernel()(dstr)


def _mm(a, w):
    return lax.dot_general(a, w, (((1,), (0,)), ((), ())),
                           preferred_element_type=jnp.float32)


def _rowmm(a, w):
    return lax.dot_general(a, w, (((1,), (1,)), ((), ())),
                           preferred_element_type=jnp.float32)


def _pack(v, tmp):
    tmp[...] = jnp.reshape(v, (PBLK, 8, HID))
    return jnp.reshape(tmp[...], (PBLK, 128))


def _unpack(v, tmp):
    tmp[...] = jnp.reshape(v, (PBLK, 8, HID))
    return jnp.reshape(tmp[...], (BLK, HID))


def _k2_body(x_ref, w1_ref, degp_ref, dinvrep_ref, a0_ref, tmp_ref):
    h0 = _rowmm(x_ref[...], w1_ref[...])
    deg = degp_ref[0] + degp_ref[1] + 1.0
    dinv = lax.rsqrt(deg)[:, None]
    dinvrep_ref[...] = _pack(jnp.broadcast_to(dinv, (BLK, HID)), tmp_ref)
    a0_ref[...] = _pack(h0 * dinv, tmp_ref)


def _k2(x, W1, degp):
    return pl.pallas_call(
        _k2_body,
        grid=(GRID,),
        in_specs=[
            pl.BlockSpec((BLK, D_IN), lambda i: (i, 0)),
            pl.BlockSpec((HID, D_IN), lambda i: (0, 0)),
            pl.BlockSpec((2, BLK), lambda i: (0, i)),
        ],
        out_specs=[
            pl.BlockSpec((PBLK, 128), lambda i: (i, 0)),
            pl.BlockSpec((PBLK, 128), lambda i: (i, 0)),
        ],
        out_shape=[
            jax.ShapeDtypeStruct((PR, 128), jnp.float32),
            jax.ShapeDtypeStruct((PR, 128), jnp.float32),
        ],
        scratch_shapes=[pltpu.VMEM((PBLK, 8, HID), jnp.float32)],
    )(x, W1, degp)


def _k3_body(p_ref, a0_ref, dinvrep_ref, b1_ref, w2k_ref, x1_ref, a1_ref):
    dr = dinvrep_ref[...]
    x1 = jnp.maximum((p_ref[0] + p_ref[1] + a0_ref[...]) * dr + b1_ref[...],
                     0.0)
    x1_ref[...] = x1
    a1_ref[...] = _mm(x1, w2k_ref[...]) * dr


def _k3(P0, a0, dinvrep, b1rep, W2K):
    return pl.pallas_call(
        _k3_body,
        grid=(GRID,),
        in_specs=[
            pl.BlockSpec((2, PBLK, 128), lambda i: (0, i, 0)),
            pl.BlockSpec((PBLK, 128), lambda i: (i, 0)),
            pl.BlockSpec((PBLK, 128), lambda i: (i, 0)),
            pl.BlockSpec((128,), lambda i: (0,)),
            pl.BlockSpec((128, 128), lambda i: (0, 0)),
        ],
        out_specs=[
            pl.BlockSpec((PBLK, 128), lambda i: (i, 0)),
            pl.BlockSpec((PBLK, 128), lambda i: (i, 0)),
        ],
        out_shape=[
            jax.ShapeDtypeStruct((PR, 128), jnp.float32),
            jax.ShapeDtypeStruct((PR, 128), jnp.float32),
        ],
    )(P0, a0, dinvrep, b1rep, W2K)


def _k4_body(p_ref, a1_ref, dinvrep_ref, b2_ref, x1_ref,
             wx1_ref, wx2_ref, bx_ref, wh_ref, wda_ref, wdb_ref, aj_ref,
             tmp_ref):
    dr = dinvrep_ref[...]
    x2p = jnp.maximum((p_ref[0] + p_ref[1] + a1_ref[...]) * dr + b2_ref[...],
                      0.0)
    x1 = _unpack(x1_ref[...], tmp_ref)
    x2 = _unpack(x2p, tmp_ref)

    G = _mm(x1, wx1_ref[...]) + _mm(x2, wx2_ref[...]) + bx_ref[...]
    cA = jax.nn.sigmoid(G[:, 0:128]) * jnp.tanh(G[:, 128:256])
    hA = jax.nn.sigmoid(G[:, 256:384]) * jnp.tanh(cA)
    GB = G[:, 384:896] + _mm(hA, wh_ref[...])
    cB = (jax.nn.sigmoid(GB[:, 128:256]) * cA
          + jax.nn.sigmoid(GB[:, 0:128]) * jnp.tanh(GB[:, 256:384]))
    hB = jax.nn.sigmoid(GB[:, 384:512]) * jnp.tanh(cB)

    ad = _mm(hA, wda_ref[...]) + _mm(hB, wdb_ref[...])
    w1 = jax.nn.sigmoid(ad)
    xj = x1 + (x2 - x1) * w1
    aj_ref[...] = _pack(xj, tmp_ref) * dr


def _k4(P1, a1, dinvrep, b2rep, x1, WX1, WX2, bX, WH, wdA, wdB):
    return pl.pallas_call(
        _k4_body,
        grid=(GRID,),
        in_specs=[
            pl.BlockSpec((2, PBLK, 128), lambda i: (0, i, 0)),
            pl.BlockSpec((PBLK, 128), lambda i: (i, 0)),
            pl.BlockSpec((PBLK, 128), lambda i: (i, 0)),
            pl.BlockSpec((128,), lambda i: (0,)),
            pl.BlockSpec((PBLK, 128), lambda i: (i, 0)),
            pl.BlockSpec((HID, 896), lambda i: (0, 0)),
            pl.BlockSpec((HID, 896), lambda i: (0, 0)),
            pl.BlockSpec((896,), lambda i: (0,)),
            pl.BlockSpec((128, 512), lambda i: (0, 0)),
            pl.BlockSpec((128, 1), lambda i: (0, 0)),
            pl.BlockSpec((128, 1), lambda i: (0, 0)),
        ],
        out_specs=pl.BlockSpec((PBLK, 128), lambda i: (i, 0)),
        out_shape=jax.ShapeDtypeStruct((PR, 128), jnp.float32),
        scratch_shapes=[pltpu.VMEM((PBLK, 8, HID), jnp.float32)],
    )(P1, a1, dinvrep, b2rep, x1, WX1, WX2, bX, WH, wdA, wdB)


def _k5_body(p_ref, aj_ref, dinvrep_ref, wlk_ref, blin_ref, out_ref,
             tmp_ref):
    xp = (p_ref[0] + p_ref[1] + aj_ref[...]) * dinvrep_ref[...]
    tmp_ref[...] = jnp.reshape(_mm(xp, wlk_ref[...]), (PBLK, 8, OUT))
    o = jnp.reshape(tmp_ref[...], (BLK, OUT)) + blin_ref[...]
    m = jnp.max(o, axis=1, keepdims=True)
    zz = o - m
    lse = jnp.log(jnp.sum(jnp.exp(zz), axis=1, keepdims=True))
    out_ref[...] = zz - lse


def _k5(Pj, aj, dinvrep, WLK, b_lin):
    return pl.pallas_call(
        _k5_body,
        grid=(GRID,),
        in_specs=[
            pl.BlockSpec((2, PBLK, 128), lambda i: (0, i, 0)),
            pl.BlockSpec((PBLK, 128), lambda i: (i, 0)),
            pl.BlockSpec((PBLK, 128), lambda i: (i, 0)),
            pl.BlockSpec((128, 8 * OUT), lambda i: (0, 0)),
            pl.BlockSpec((OUT,), lambda i: (0,)),
        ],
        out_specs=pl.BlockSpec((BLK, OUT), lambda i: (i, 0)),
        out_shape=jax.ShapeDtypeStruct((N, OUT), jnp.float32),
        scratch_shapes=[pltpu.VMEM((PBLK, 8, OUT), jnp.float32)],
    )(Pj, aj, dinvrep, WLK, b_lin)


def _k4_weights(W_ih_f, W_hh_f, b_ih_f, b_hh_f, W_ih_r, W_hh_r, b_ih_r, b_hh_r,
                W_att):
    Tf, Tr = W_ih_f.T, W_ih_r.T
    Uf, Ur = W_hh_f.T, W_hh_r.T
    bsf = b_ih_f + b_hh_f
    bsr = b_ih_r + b_hh_r
    gi, gf, gg, go = (slice(32 * k, 32 * (k + 1)) for k in range(4))
    Z16 = jnp.zeros((HID, 32), jnp.float32)
    Z16w = jnp.zeros((HID, 64), jnp.float32)

    def xg(fwd, rev):
        return jnp.concatenate([fwd if fwd is not None else Z16,
                                rev if rev is not None else Z16, Z16w], axis=1)

    WX1 = jnp.concatenate(
        [xg(Tf[:, gi], None), xg(Tf[:, gg], None), xg(Tf[:, go], None),
         xg(None, Tr[:, gi]), xg(None, Tr[:, gf]), xg(None, Tr[:, gg]),
         xg(None, Tr[:, go])], axis=1)
    WX2 = jnp.concatenate(
        [xg(None, Tr[:, gi]), xg(None, Tr[:, gg]), xg(None, Tr[:, go]),
         xg(Tf[:, gi], None), xg(Tf[:, gf], None), xg(Tf[:, gg], None),
         xg(Tf[:, go], None)], axis=1)
    z64 = jnp.zeros((64,), jnp.float32)

    def bg(gsl):
        return jnp.concatenate([bsf[gsl], bsr[gsl], z64])

    bX = jnp.concatenate([bg(gi), bg(gg), bg(go), bg(gi), bg(gf), bg(gg),
                          bg(go)])
    Z3296 = jnp.zeros((LSTM_H, 96), jnp.float32)
    Z3232 = jnp.zeros((LSTM_H, 32), jnp.float32)
    band_f = jnp.concatenate(
        [jnp.concatenate([Uf[:, g], Z3296], axis=1) for g in (gi, gf, gg, go)],
        axis=1)
    band_r = jnp.concatenate(
        [jnp.concatenate([Z3232, Ur[:, g], Z3296[:, :64]], axis=1)
         for g in (gi, gf, gg, go)], axis=1)
    WH = jnp.concatenate([band_f, band_r,
                          jnp.zeros((64, 512), jnp.float32)], axis=0)
    w = W_att[0]
    wdA = jnp.concatenate([-w[0:32], w[32:64], z64])[:, None]
    wdB = jnp.concatenate([w[0:32], -w[32:64], z64])[:, None]
    return WX1, WX2, bX, WH, wdA, wdB


def kernel(x, edge_index, W1, b1, W2, b2, W_ih_f, W_hh_f, b_ih_f, b_hh_f,
           W_ih_r, W_hh_r, b_ih_r, b_hh_r, W_att, b_att, W_lin, b_lin):
    src = edge_index[0]
    dst = edge_index[1]
    pad_src = jnp.zeros((E_PAD - E,), jnp.int32)
    pad_dst = jnp.full((E_PAD - E,), N_PAD - 1, jnp.int32)
    srcr = jnp.concatenate([src, pad_src]).reshape(R_TOT, LANES)
    dstr = jnp.concatenate([dst, pad_dst]).reshape(R_TOT, LANES)

    degp = _deg_call(dstr)
    dinvrep, a0 = _k2(x, W1, degp)
    P0 = _prop_call(a0.reshape(N_PAD, HID), srcr, dstr)
    x1, a1 = _k3(P0.reshape(2, PR, 128), a0, dinvrep,
                 jnp.tile(b1, 8), jnp.kron(jnp.eye(8, dtype=jnp.float32), W2.T))
    P1 = _prop_call(a1.reshape(N_PAD, HID), srcr, dstr)
    WX1, WX2, bX, WH, wdA, wdB = _k4_weights(
        W_ih_f, W_hh_f, b_ih_f, b_hh_f, W_ih_r, W_hh_r, b_ih_r, b_hh_r, W_att)
    aj = _k4(P1.reshape(2, PR, 128), a1, dinvrep, jnp.tile(b2, 8), x1,
             WX1, WX2, bX, WH, wdA, wdB)
    Pj = _prop_call(aj.reshape(N_PAD, HID), srcr, dstr)
    return _k5(Pj.reshape(2, PR, 128), aj, dinvrep,
               jnp.kron(jnp.eye(8, dtype=jnp.float32), W_lin.T), b_lin)


# speedup vs baseline: 1.7449x; 1.0121x over previous
import functools

import jax
import jax.numpy as jnp
from jax import lax
from jax.experimental import pallas as pl
from jax.experimental.pallas import tpu as pltpu
from jax.experimental.pallas import tpu_sc as plsc

N = 100000
E = 1600000
D_IN = 128
HID = 16
OUT = 64
LSTM_H = 32

NW = 32
LANES = 128
KB = 4
DKB = 8
RPW = 392
R_TOT = NW * RPW
E_PAD = R_TOT * LANES
NB = RPW // KB
DNB = RPW // DKB
N_PAD = 100352
STRIPE = N_PAD // 16
ZR = 112

BLK = 4096
GRID = -(-N // BLK)
PR = N_PAD // 8
PBLK = BLK // 8

def _prop_body(h_hbm, src_hbm, dst_hbm, out_hbm,
               sv0, dv0, rv0, sv1, dv1, rv1, zv, acc,
               isem0, isem1, gsem0, gsem1, ssem0, ssem1):
    c = lax.axis_index("c")
    s = lax.axis_index("s")
    w = s * 2 + c
    base = s * STRIPE

    sv = (sv0, sv1)
    dv = (dv0, dv1)
    rv = (rv0, rv1)
    isem = (isem0, isem1)
    gsem = (gsem0, gsem1)
    ssem = (ssem0, ssem1)

    def zrow(i, carry):
        zv[i] = jnp.zeros((HID,), jnp.float32)
        return carry

    lax.fori_loop(0, ZR, zrow, 0)
    for r in range(STRIPE // ZR):
        pltpu.sync_copy(zv, acc.at[pl.ds(base + r * ZR, ZR)])
    plsc.subcore_barrier()

    row0 = w * RPW

    def idx_start(i, b):
        rbase = row0 + i * KB
        pltpu.async_copy(src_hbm.at[pl.ds(rbase, KB)], sv[b], isem[b])
        pltpu.async_copy(dst_hbm.at[pl.ds(rbase, KB)], dv[b], isem[b])

    def idx_wait(b):
        pltpu.make_async_copy(src_hbm.at[pl.ds(0, KB)], sv[b], isem[b]).wait()
        pltpu.make_async_copy(dst_hbm.at[pl.ds(0, KB)], dv[b], isem[b]).wait()

    def gat_start(b):
        for j in range(KB):
            pltpu.async_copy(h_hbm.at[sv[b].at[j]], rv[b].at[j], gsem[b])

    def gat_wait(b):
        for j in range(KB):
            pltpu.make_async_copy(h_hbm.at[sv[b].at[j]], rv[b].at[j],
                                  gsem[b]).wait()

    def sca_start(b):
        for j in range(KB):
            pltpu.async_copy(rv[b].at[j], acc.at[dv[b].at[j]], ssem[b],
                             add=True)

    def sca_wait(b):
        for j in range(KB):
            pltpu.make_async_copy(rv[b].at[j], acc.at[dv[b].at[j]],
                                  ssem[b]).wait()

    def step(i, b, first):
        if not first:
            sca_wait(b)
        idx_start(i, b)
        gat_wait(b ^ 1)
        sca_start(b ^ 1)
        idx_wait(b)
        gat_start(b)

    idx_start(0, 0)
    idx_wait(0)
    gat_start(0)
    step(1, 1, True)

    def pair(j, carry):
        step(2 + 2 * j, 0, False)
        step(3 + 2 * j, 1, False)
        return carry

    lax.fori_loop(0, (NB - 2) // 2, pair, 0)

    gat_wait(1)
    sca_start(1)
    sca_wait(0)
    sca_wait(1)
    plsc.subcore_barrier()
    pltpu.sync_copy(acc.at[pl.ds(base, STRIPE)], out_hbm.at[c, pl.ds(base, STRIPE)])


def _deg_body(dst_hbm, out_hbm, dst_v, ones_v, zv, acc):
    c = lax.axis_index("c")
    s = lax.axis_index("s")
    w = s * 2 + c
    base = s * STRIPE

    def zchunk(i, carry):
        zv[pl.ds(i * 16, 16)] = jnp.zeros((16,), jnp.float32)
        return carry

    lax.fori_loop(0, ZR // 16, zchunk, 0)
    for j in range(LANES // 16):
        ones_v[pl.ds(j * 16, 16)] = jnp.ones((16,), jnp.float32)
    for r in range(STRIPE // ZR):
        pltpu.sync_copy(zv, acc.at[pl.ds(base + r * ZR, ZR)])
    plsc.subcore_barrier()

    row0 = w * RPW

    def block(bi, carry):
        rbase = row0 + bi * DKB
        pltpu.sync_copy(dst_hbm.at[pl.ds(rbase, DKB)], dst_v)
        for j in range(DKB):
            pltpu.sync_copy(ones_v, acc.at[dst_v.at[j]], add=True)
        return carry

    lax.fori_loop(0, DNB, block, 0)
    plsc.subcore_barrier()
    pltpu.sync_copy(acc.at[pl.ds(base, STRIPE)], out_hbm.at[c, pl.ds(base, STRIPE)])


@functools.lru_cache(maxsize=None)
def _prop_kernel():
    mesh = plsc.VectorSubcoreMesh(core_axis_name="c", subcore_axis_name="s")
    return pl.kernel(
        _prop_body,
        mesh=mesh,
        compiler_params=pltpu.CompilerParams(use_tc_tiling_on_sc=False),
        out_type=jax.ShapeDtypeStruct((2, N_PAD, HID), jnp.float32),
        scratch_types=[
            pltpu.VMEM((KB, LANES), jnp.int32),
            pltpu.VMEM((KB, LANES), jnp.int32),
            pltpu.VMEM((KB, LANES, HID), jnp.float32),
            pltpu.VMEM((KB, LANES), jnp.int32),
            pltpu.VMEM((KB, LANES), jnp.int32),
            pltpu.VMEM((KB, LANES, HID), jnp.float32),
            pltpu.VMEM((ZR, HID), jnp.float32),
            pltpu.VMEM_SHARED((N_PAD, HID), jnp.float32),
            pltpu.SemaphoreType.DMA,
            pltpu.SemaphoreType.DMA,
            pltpu.SemaphoreType.DMA,
            pltpu.SemaphoreType.DMA,
            pltpu.SemaphoreType.DMA,
            pltpu.SemaphoreType.DMA,
        ],
    )


@functools.lru_cache(maxsize=None)
def _deg_kernel():
    mesh = plsc.VectorSubcoreMesh(core_axis_name="c", subcore_axis_name="s")
    return pl.kernel(
        _deg_body,
        mesh=mesh,
        compiler_params=pltpu.CompilerParams(use_tc_tiling_on_sc=False),
        out_type=jax.ShapeDtypeStruct((2, N_PAD), jnp.float32),
        scratch_types=[
            pltpu.VMEM((DKB, LANES), jnp.int32),
            pltpu.VMEM((LANES,), jnp.float32),
            pltpu.VMEM((ZR,), jnp.float32),
            pltpu.VMEM_SHARED((N_PAD,), jnp.float32),
        ],
    )


def _prop_call(h, srcr, dstr):
    return _prop_kernel()(h, srcr, dstr)


def _deg_call(dstr):
    return _deg_kernel()(dstr)


def _mm(a, w):
    return lax.dot_general(a, w, (((1,), (0,)), ((), ())),
                           preferred_element_type=jnp.float32)


def _rowmm(a, w):
    return lax.dot_general(a, w, (((1,), (1,)), ((), ())),
                           preferred_element_type=jnp.float32)


def _pack(v, tmp):
    tmp[...] = jnp.reshape(v, (PBLK, 8, HID))
    return jnp.reshape(tmp[...], (PBLK, 128))


def _unpack(v, tmp):
    tmp[...] = jnp.reshape(v, (PBLK, 8, HID))
    return jnp.reshape(tmp[...], (BLK, HID))


def _k2a_body(x_ref, w1_ref, h0p_ref, tmp_ref):
    h0p_ref[...] = _pack(_rowmm(x_ref[...], w1_ref[...]), tmp_ref)


def _k2a(x, W1):
    return pl.pallas_call(
        _k2a_body,
        grid=(GRID,),
        in_specs=[
            pl.BlockSpec((BLK, D_IN), lambda i: (i, 0)),
            pl.BlockSpec((HID, D_IN), lambda i: (0, 0)),
        ],
        out_specs=pl.BlockSpec((PBLK, 128), lambda i: (i, 0)),
        out_shape=jax.ShapeDtypeStruct((PR, 128), jnp.float32),
        scratch_shapes=[pltpu.VMEM((PBLK, 8, HID), jnp.float32)],
    )(x, W1)


def _k2b_body(degp_ref, h0p_ref, dinvrep_ref, a0_ref, tmp_ref):
    deg = degp_ref[0] + degp_ref[1] + 1.0
    dinv = lax.rsqrt(deg)[:, None]
    dr = _pack(jnp.broadcast_to(dinv, (BLK, HID)), tmp_ref)
    dinvrep_ref[...] = dr
    a0_ref[...] = h0p_ref[...] * dr


def _k2b(degp, h0p):
    return pl.pallas_call(
        _k2b_body,
        grid=(GRID,),
        in_specs=[
            pl.BlockSpec((2, BLK), lambda i: (0, i)),
            pl.BlockSpec((PBLK, 128), lambda i: (i, 0)),
        ],
        out_specs=[
            pl.BlockSpec((PBLK, 128), lambda i: (i, 0)),
            pl.BlockSpec((PBLK, 128), lambda i: (i, 0)),
        ],
        out_shape=[
            jax.ShapeDtypeStruct((PR, 128), jnp.float32),
            jax.ShapeDtypeStruct((PR, 128), jnp.float32),
        ],
        scratch_shapes=[pltpu.VMEM((PBLK, 8, HID), jnp.float32)],
    )(degp, h0p)


def _k3_body(p_ref, a0_ref, dinvrep_ref, b1_ref, w2k_ref, x1_ref, a1_ref):
    dr = dinvrep_ref[...]
    x1 = jnp.maximum((p_ref[0] + p_ref[1] + a0_ref[...]) * dr + b1_ref[...],
                     0.0)
    x1_ref[...] = x1
    a1_ref[...] = _mm(x1, w2k_ref[...]) * dr


def _k3(P0, a0, dinvrep, b1rep, W2K):
    return pl.pallas_call(
        _k3_body,
        grid=(GRID,),
        in_specs=[
            pl.BlockSpec((2, PBLK, 128), lambda i: (0, i, 0)),
            pl.BlockSpec((PBLK, 128), lambda i: (i, 0)),
            pl.BlockSpec((PBLK, 128), lambda i: (i, 0)),
            pl.BlockSpec((128,), lambda i: (0,)),
            pl.BlockSpec((128, 128), lambda i: (0, 0)),
        ],
        out_specs=[
            pl.BlockSpec((PBLK, 128), lambda i: (i, 0)),
            pl.BlockSpec((PBLK, 128), lambda i: (i, 0)),
        ],
        out_shape=[
            jax.ShapeDtypeStruct((PR, 128), jnp.float32),
            jax.ShapeDtypeStruct((PR, 128), jnp.float32),
        ],
    )(P0, a0, dinvrep, b1rep, W2K)


def _k4_body(p_ref, a1_ref, dinvrep_ref, b2_ref, x1_ref,
             wx1_ref, wx2_ref, bx_ref, wh_ref, wda_ref, wdb_ref, aj_ref,
             tmp_ref):
    dr = dinvrep_ref[...]
    x2p = jnp.maximum((p_ref[0] + p_ref[1] + a1_ref[...]) * dr + b2_ref[...],
                      0.0)
    x1 = _unpack(x1_ref[...], tmp_ref)
    x2 = _unpack(x2p, tmp_ref)

    G = _mm(x1, wx1_ref[...]) + _mm(x2, wx2_ref[...]) + bx_ref[...]
    cA = jax.nn.sigmoid(G[:, 0:128]) * jnp.tanh(G[:, 128:256])
    hA = jax.nn.sigmoid(G[:, 256:384]) * jnp.tanh(cA)
    GB = G[:, 384:896] + _mm(hA, wh_ref[...])
    cB = (jax.nn.sigmoid(GB[:, 128:256]) * cA
          + jax.nn.sigmoid(GB[:, 0:128]) * jnp.tanh(GB[:, 256:384]))
    hB = jax.nn.sigmoid(GB[:, 384:512]) * jnp.tanh(cB)

    ad = _mm(hA, wda_ref[...]) + _mm(hB, wdb_ref[...])
    w1 = jax.nn.sigmoid(ad)
    xj = x1 + (x2 - x1) * w1
    aj_ref[...] = _pack(xj, tmp_ref) * dr


def _k4(P1, a1, dinvrep, b2rep, x1, WX1, WX2, bX, WH, wdA, wdB):
    return pl.pallas_call(
        _k4_body,
        grid=(GRID,),
        in_specs=[
            pl.BlockSpec((2, PBLK, 128), lambda i: (0, i, 0)),
            pl.BlockSpec((PBLK, 128), lambda i: (i, 0)),
            pl.BlockSpec((PBLK, 128), lambda i: (i, 0)),
            pl.BlockSpec((128,), lambda i: (0,)),
            pl.BlockSpec((PBLK, 128), lambda i: (i, 0)),
            pl.BlockSpec((HID, 896), lambda i: (0, 0)),
            pl.BlockSpec((HID, 896), lambda i: (0, 0)),
            pl.BlockSpec((896,), lambda i: (0,)),
            pl.BlockSpec((128, 512), lambda i: (0, 0)),
            pl.BlockSpec((128, 1), lambda i: (0, 0)),
            pl.BlockSpec((128, 1), lambda i: (0, 0)),
        ],
        out_specs=pl.BlockSpec((PBLK, 128), lambda i: (i, 0)),
        out_shape=jax.ShapeDtypeStruct((PR, 128), jnp.float32),
        scratch_shapes=[pltpu.VMEM((PBLK, 8, HID), jnp.float32)],
    )(P1, a1, dinvrep, b2rep, x1, WX1, WX2, bX, WH, wdA, wdB)


def _k5_body(p_ref, aj_ref, dinvrep_ref, wlk_ref, blin_ref, out_ref,
             tmp_ref):
    xp = (p_ref[0] + p_ref[1] + aj_ref[...]) * dinvrep_ref[...]
    tmp_ref[...] = jnp.reshape(_mm(xp, wlk_ref[...]), (PBLK, 8, OUT))
    o = jnp.reshape(tmp_ref[...], (BLK, OUT)) + blin_ref[...]
    m = jnp.max(o, axis=1, keepdims=True)
    zz = o - m
    lse = jnp.log(jnp.sum(jnp.exp(zz), axis=1, keepdims=True))
    out_ref[...] = zz - lse


def _k5(Pj, aj, dinvrep, WLK, b_lin):
    return pl.pallas_call(
        _k5_body,
        grid=(GRID,),
        in_specs=[
            pl.BlockSpec((2, PBLK, 128), lambda i: (0, i, 0)),
            pl.BlockSpec((PBLK, 128), lambda i: (i, 0)),
            pl.BlockSpec((PBLK, 128), lambda i: (i, 0)),
            pl.BlockSpec((128, 8 * OUT), lambda i: (0, 0)),
            pl.BlockSpec((OUT,), lambda i: (0,)),
        ],
        out_specs=pl.BlockSpec((BLK, OUT), lambda i: (i, 0)),
        out_shape=jax.ShapeDtypeStruct((N, OUT), jnp.float32),
        scratch_shapes=[pltpu.VMEM((PBLK, 8, OUT), jnp.float32)],
    )(Pj, aj, dinvrep, WLK, b_lin)


def _k4_weights(W_ih_f, W_hh_f, b_ih_f, b_hh_f, W_ih_r, W_hh_r, b_ih_r, b_hh_r,
                W_att):
    Tf, Tr = W_ih_f.T, W_ih_r.T
    Uf, Ur = W_hh_f.T, W_hh_r.T
    bsf = b_ih_f + b_hh_f
    bsr = b_ih_r + b_hh_r
    gi, gf, gg, go = (slice(32 * k, 32 * (k + 1)) for k in range(4))
    Z16 = jnp.zeros((HID, 32), jnp.float32)
    Z16w = jnp.zeros((HID, 64), jnp.float32)

    def xg(fwd, rev):
        return jnp.concatenate([fwd if fwd is not None else Z16,
                                rev if rev is not None else Z16, Z16w], axis=1)

    WX1 = jnp.concatenate(
        [xg(Tf[:, gi], None), xg(Tf[:, gg], None), xg(Tf[:, go], None),
         xg(None, Tr[:, gi]), xg(None, Tr[:, gf]), xg(None, Tr[:, gg]),
         xg(None, Tr[:, go])], axis=1)
    WX2 = jnp.concatenate(
        [xg(None, Tr[:, gi]), xg(None, Tr[:, gg]), xg(None, Tr[:, go]),
         xg(Tf[:, gi], None), xg(Tf[:, gf], None), xg(Tf[:, gg], None),
         xg(Tf[:, go], None)], axis=1)
    z64 = jnp.zeros((64,), jnp.float32)

    def bg(gsl):
        return jnp.concatenate([bsf[gsl], bsr[gsl], z64])

    bX = jnp.concatenate([bg(gi), bg(gg), bg(go), bg(gi), bg(gf), bg(gg),
                          bg(go)])
    Z3296 = jnp.zeros((LSTM_H, 96), jnp.float32)
    Z3232 = jnp.zeros((LSTM_H, 32), jnp.float32)
    band_f = jnp.concatenate(
        [jnp.concatenate([Uf[:, g], Z3296], axis=1) for g in (gi, gf, gg, go)],
        axis=1)
    band_r = jnp.concatenate(
        [jnp.concatenate([Z3232, Ur[:, g], Z3296[:, :64]], axis=1)
         for g in (gi, gf, gg, go)], axis=1)
    WH = jnp.concatenate([band_f, band_r,
                          jnp.zeros((64, 512), jnp.float32)], axis=0)
    w = W_att[0]
    wdA = jnp.concatenate([-w[0:32], w[32:64], z64])[:, None]
    wdB = jnp.concatenate([w[0:32], -w[32:64], z64])[:, None]
    return WX1, WX2, bX, WH, wdA, wdB


def kernel(x, edge_index, W1, b1, W2, b2, W_ih_f, W_hh_f, b_ih_f, b_hh_f,
           W_ih_r, W_hh_r, b_ih_r, b_hh_r, W_att, b_att, W_lin, b_lin):
    src = edge_index[0]
    dst = edge_index[1]
    pad_src = jnp.zeros((E_PAD - E,), jnp.int32)
    pad_dst = jnp.full((E_PAD - E,), N_PAD - 1, jnp.int32)
    srcr = jnp.concatenate([src, pad_src]).reshape(R_TOT, LANES)
    dstr = jnp.concatenate([dst, pad_dst]).reshape(R_TOT, LANES)

    h0p = _k2a(x, W1)
    degp = _deg_call(dstr)
    dinvrep, a0 = _k2b(degp, h0p)
    P0 = _prop_call(a0.reshape(N_PAD, HID), srcr, dstr)
    x1, a1 = _k3(P0.reshape(2, PR, 128), a0, dinvrep,
                 jnp.tile(b1, 8), jnp.kron(jnp.eye(8, dtype=jnp.float32), W2.T))
    P1 = _prop_call(a1.reshape(N_PAD, HID), srcr, dstr)
    WX1, WX2, bX, WH, wdA, wdB = _k4_weights(
        W_ih_f, W_hh_f, b_ih_f, b_hh_f, W_ih_r, W_hh_r, b_ih_r, b_hh_r, W_att)
    aj = _k4(P1.reshape(2, PR, 128), a1, dinvrep, jnp.tile(b2, 8), x1,
             WX1, WX2, bX, WH, wdA, wdB)
    Pj = _prop_call(aj.reshape(N_PAD, HID), srcr, dstr)
    return _k5(Pj.reshape(2, PR, 128), aj, dinvrep,
               jnp.kron(jnp.eye(8, dtype=jnp.float32), W_lin.T), b_lin)
